# baseline (speedup 1.0000x reference)
import functools

import jax
import jax.numpy as jnp
from jax import lax
from jax.experimental import pallas as pl
from jax.experimental.pallas import tpu as pltpu
from jax.experimental.pallas import tpu_sc as plsc

N = 10000
E = 160000
D = 256
NW = 32
NPT = 313
NPAD = NW * NPT
CAP = 16384
CHK = 8000
EB = 80

_mesh = plsc.VectorSubcoreMesh(core_axis_name="c", subcore_axis_name="s")
_sc_params = pltpu.CompilerParams(needs_layout_passes=False)


def _wid():
    return lax.axis_index("s") * 2 + lax.axis_index("c")


@functools.partial(
    pl.kernel,
    out_type=(
        jax.ShapeDtypeStruct((NW * CAP,), jnp.int32),
        jax.ShapeDtypeStruct((NW * CAP,), jnp.int32),
        jax.ShapeDtypeStruct((NW * CAP,), jnp.float32),
        jax.ShapeDtypeStruct((NW * 16,), jnp.int32),
    ),
    mesh=_mesh,
    scratch_types=[
        pltpu.VMEM((CHK,), jnp.int32),
        pltpu.VMEM((CHK,), jnp.int32),
        pltpu.VMEM((CHK,), jnp.float32),
        pltpu.VMEM((CAP + 16,), jnp.int32),
        pltpu.VMEM((CAP + 16,), jnp.int32),
        pltpu.VMEM((CAP + 16,), jnp.float32),
        pltpu.VMEM((CAP + 16,), jnp.int32),
        pltpu.VMEM((CAP + 16,), jnp.int32),
        pltpu.VMEM((CAP + 16,), jnp.float32),
        pltpu.VMEM((336,), jnp.int32),
        pltpu.VMEM((336,), jnp.int32),
        pltpu.VMEM((16,), jnp.int32),
    ],
    compiler_params=_sc_params,
)
def _route(src, dst, dist, src_o, off_o, dist_o, cnt_o,
           dst_c, src_c, dist_c, lsrc, loff, ldist, ssrc, soff, sdist,
           hist, start, cvec):
    wid = _wid()
    lo = wid * NPT
    iota = lax.iota(jnp.int32, 16)

    def chunk(c, ptr):
        base = c * CHK
        pltpu.sync_copy(dst.at[pl.ds(base, CHK)], dst_c)
        pltpu.sync_copy(src.at[pl.ds(base, CHK)], src_c)
        pltpu.sync_copy(dist.at[pl.ds(base, CHK)], dist_c)

        def group(j, p):
            d = dst_c[pl.ds(j * 16, 16)]
            m = (d >= lo) & (d < lo + NPT)
            key = jnp.where(m, iota, iota + 16)
            packed = src_c[pl.ds(j * 16, 16)] * 512 + (d - lo)
            _, po = plsc.sort_key_val(key, packed)
            _, di = plsc.sort_key_val(key, dist_c[pl.ds(j * 16, 16)])
            pos = jnp.minimum(p + iota, CAP + 15)
            plsc.store_scatter(lsrc, [pos], lax.shift_right_logical(po, 9))
            plsc.store_scatter(loff, [pos], po & 511)
            plsc.store_scatter(ldist, [pos], di)
            cnt = plsc.all_reduce_population_count(m)[0]
            return p + cnt

        return lax.fori_loop(0, CHK // 16, group, ptr, unroll=2)

    ptr = lax.fori_loop(0, E // CHK, chunk, jnp.int32(0))
    n = jnp.minimum(ptr, CAP - 64)

    zero16 = jnp.zeros((16,), jnp.int32)
    for g in range(336 // 16):
        hist[pl.ds(g * 16, 16)] = zero16
    onehot = jnp.where(iota == 0, 1, 0).astype(jnp.int32)

    def bhist(i, _):
        o = loff[pl.ds(i, 16)][0]
        plsc.addupdate(hist.at[pl.ds(o, 16)], onehot)
        return 0

    lax.fori_loop(0, n, bhist, 0)

    def bpre(g, carry):
        v = hist[pl.ds(g * 16, 16)]
        c = plsc.cumsum(v)
        start[pl.ds(g * 16, 16)] = c - v + carry
        return carry + c[15]

    lax.fori_loop(0, 336 // 16, bpre, jnp.int32(0))

    def bscat(i, _):
        o = loff[pl.ds(i, 16)][0]
        sv = start[pl.ds(o, 16)]
        p = sv[0]
        s = lsrc[pl.ds(i, 16)][0]
        dd = ldist[pl.ds(i, 16)][0]
        vs = ssrc[pl.ds(p, 16)]
        ssrc[pl.ds(p, 16)] = jnp.where(iota == 0, s, vs)
        vo = soff[pl.ds(p, 16)]
        soff[pl.ds(p, 16)] = jnp.where(iota == 0, o * D, vo)
        vd = sdist[pl.ds(p, 16)]
        sdist[pl.ds(p, 16)] = jnp.where(iota == 0, dd, vd)
        plsc.addupdate(start.at[pl.ds(o, 16)], onehot)
        return 0

    lax.fori_loop(0, n, bscat, 0)

    for g in range(128 // 16):
        pos = jnp.minimum(n + g * 16 + iota, CAP + 15)
        plsc.store_scatter(ssrc, [pos], jnp.zeros((16,), jnp.int32))
        plsc.store_scatter(soff, [pos], jnp.full((16,), NPT * D, jnp.int32))
        plsc.store_scatter(sdist, [pos], jnp.zeros((16,), jnp.float32))
    pltpu.sync_copy(ssrc.at[pl.ds(0, CAP)], src_o.at[pl.ds(wid * CAP, CAP)])
    pltpu.sync_copy(soff.at[pl.ds(0, CAP)], off_o.at[pl.ds(wid * CAP, CAP)])
    pltpu.sync_copy(sdist.at[pl.ds(0, CAP)], dist_o.at[pl.ds(wid * CAP, CAP)])
    cvec[...] = jnp.zeros((16,), jnp.int32) + n
    pltpu.sync_copy(cvec, cnt_o.at[pl.ds(wid * 16, 16)])


@functools.partial(
    pl.kernel,
    out_type=jax.ShapeDtypeStruct((NPAD * D,), jnp.float32),
    mesh=_mesh,
    scratch_types=[
        pltpu.VMEM(((NPT + 1) * D,), jnp.float32),
        pltpu.VMEM((D,), jnp.float32),
        pltpu.VMEM((2, EB), jnp.int32),
        pltpu.VMEM((2, EB + 16), jnp.int32),
        pltpu.VMEM((2, EB + 16), jnp.float32),
        pltpu.VMEM((2, EB, D), jnp.float32),
        pltpu.VMEM((16,), jnp.int32),
        pltpu.VMEM((D,), jnp.float32),
        pltpu.VMEM((16,), jnp.int32),
        pltpu.SemaphoreType.DMA,
        pltpu.SemaphoreType.DMA,
        pltpu.SemaphoreType.DMA,
        pltpu.SemaphoreType.DMA,
    ],
    compiler_params=_sc_params,
)
def _edge(a_hbm, wrow, src_l, off_l, dist_l, cnt_l, out,
          acc, wv, idx, offc, dsc, rows, cvec, rbuf, pvbuf,
          sl0, sl1, sg0, sg1):
    wid = _wid()
    seml = (sl0, sl1)
    semg = (sg0, sg1)

    def fill(i, _):
        acc[pl.ds(i * 16, 16)] = jnp.full((16,), jnp.inf, jnp.float32)
        return 0

    lax.fori_loop(0, (NPT + 1) * D // 16, fill, 0)
    pltpu.sync_copy(wrow, wv)
    pltpu.sync_copy(cnt_l.at[pl.ds(wid * 16, 16)], cvec)
    n = jnp.minimum(cvec[...][0], CAP - 64)
    nb = (n + (EB - 1)) // EB

    def issue_l(b, par):
        pltpu.async_copy(src_l.at[pl.ds(wid * CAP + b * EB, EB)], idx.at[par],
                         seml[par])
        pltpu.async_copy(off_l.at[pl.ds(wid * CAP + b * EB, EB)],
                         offc.at[par, pl.ds(0, EB)], seml[par])
        pltpu.async_copy(dist_l.at[pl.ds(wid * CAP + b * EB, EB)],
                         dsc.at[par, pl.ds(0, EB)], seml[par])

    def wait_l(b, par):
        pltpu.make_async_copy(src_l.at[pl.ds(wid * CAP + b * EB, EB)], idx.at[par],
                              seml[par]).wait()
        pltpu.make_async_copy(off_l.at[pl.ds(wid * CAP + b * EB, EB)],
                              offc.at[par, pl.ds(0, EB)], seml[par]).wait()
        pltpu.make_async_copy(dist_l.at[pl.ds(wid * CAP + b * EB, EB)],
                              dsc.at[par, pl.ds(0, EB)], seml[par]).wait()

    def issue_g(par):
        pltpu.async_copy(a_hbm.at[idx.at[par]], rows.at[par], semg[par])

    def wait_g(par):
        pltpu.make_async_copy(a_hbm.at[idx.at[par]], rows.at[par],
                              semg[par]).wait()

    @pl.when(nb > 0)
    def _():
        issue_l(0, 0)
        wait_l(0, 0)
        issue_g(0)

    @pl.when(nb > 1)
    def _():
        issue_l(1, 1)

    wvs = tuple(wv[pl.ds(g * 16, 16)] for g in range(D // 16))
    iota = lax.iota(jnp.int32, 16)
    inf16 = jnp.full((16,), jnp.inf, jnp.float32)
    pvbuf[...] = jnp.full((16,), -1, jnp.int32)
    zf = jnp.zeros((16,), jnp.float32)
    for g in range(D // 16):
        rbuf[pl.ds(g * 16, 16)] = zf

    def pair(p, carry):
        for q in range(2):
            par = q
            nxt = 1 - q
            bi = p * 2 + q

            @pl.when(bi < nb)
            def _():
                wait_g(par)

                @pl.when(bi + 1 < nb)
                def _():
                    wait_l(bi + 1, nxt)
                    issue_g(nxt)

                po0 = pvbuf[...]
                rm0 = tuple(rbuf[pl.ds(g * 16, 16)] for g in range(D // 16))

                def edge(i, st):
                    po, rm = st
                    gb = (i // 16) * 16
                    jv = jnp.zeros((16,), jnp.int32) + (i - gb)
                    offv = offc[par, pl.ds(gb, 16)]
                    distv = dsc[par, pl.ds(gb, 16)]
                    base = offv.at[jv].get(mode="promise_in_bounds")
                    dv = distv.at[jv].get(mode="promise_in_bounds")
                    msk = base == po
                    a0 = base + iota
                    nm = []
                    for g in range(D // 16):
                        r = rows[par, i, pl.ds(g * 16, 16)]
                        prev = jnp.where(msk, rm[g], inf16)
                        v = jnp.minimum(r + dv * carry[g], prev)
                        plsc.store_scatter(acc, [a0 + (g * 16)], v)
                        nm.append(v)
                    return (base, tuple(nm))

                po1, rm1 = lax.fori_loop(0, EB, edge, (po0, rm0))
                pvbuf[...] = po1
                for g in range(D // 16):
                    rbuf[pl.ds(g * 16, 16)] = rm1[g]

                @pl.when(bi + 2 < nb)
                def _():
                    issue_l(bi + 2, par)

        return carry

    lax.fori_loop(0, (nb + 1) // 2, pair, wvs)
    pltpu.sync_copy(acc.at[pl.ds(0, NPT * D)],
                    out.at[pl.ds(wid * NPT * D, NPT * D)])


_BLK = 2000


def _leaky(x):
    return jnp.where(x > 0, x, 0.01 * x)


def _project_body(h_ref, w_ref, o_ref):
    o_ref[...] = jnp.dot(h_ref[...], w_ref[...],
                         preferred_element_type=jnp.float32)


def _project(h, w):
    return pl.pallas_call(
        _project_body,
        grid=(N // _BLK,),
        in_specs=[
            pl.BlockSpec((_BLK, D), lambda i: (i, 0)),
            pl.BlockSpec((D, D), lambda i: (0, 0)),
        ],
        out_specs=pl.BlockSpec((_BLK, D), lambda i: (i, 0)),
        out_shape=jax.ShapeDtypeStruct((N, D), jnp.float32),
    )(h, w)


def _node_body(h_ref, s_ref, wd_ref, wn_ref, o_ref):
    h = h_ref[...]
    t = s_ref[...] + jnp.dot(h, wd_ref[...], preferred_element_type=jnp.float32)
    red = jnp.where(jnp.isfinite(t), _leaky(t), 0.0)
    wn = wn_ref[...]
    o = (jnp.dot(h, wn[:D], preferred_element_type=jnp.float32)
         + jnp.dot(red, wn[D:], preferred_element_type=jnp.float32))
    o_ref[...] = _leaky(o)


def _node(h, s, wd, wn):
    return pl.pallas_call(
        _node_body,
        grid=(N // _BLK,),
        in_specs=[
            pl.BlockSpec((_BLK, D), lambda i: (i, 0)),
            pl.BlockSpec((_BLK, D), lambda i: (i, 0)),
            pl.BlockSpec((D, D), lambda i: (0, 0)),
            pl.BlockSpec((2 * D, D), lambda i: (0, 0)),
        ],
        out_specs=pl.BlockSpec((_BLK, D), lambda i: (i, 0)),
        out_shape=jax.ShapeDtypeStruct((N, D), jnp.float32),
    )(h, s, wd, wn)


def kernel(nf, edge_index, dist, We0, Wn0, We1, Wn1, We2, Wn2):
    src_l, off_l, dist_l, cnt_l = _route(edge_index[0], edge_index[1], dist)
    h = nf
    for We, Wn in ((We0, Wn0), (We1, Wn1), (We2, Wn2)):
        a = _project(h, We[1:D + 1])
        s_flat = _edge(a, We[0], src_l, off_l, dist_l, cnt_l)
        s = s_flat.reshape(NPAD, D)[:N]
        h = _node(h, s, We[D + 1:], Wn)
    return h


# speedup vs baseline: 1.3091x; 1.3091x over previous
import functools

import jax
import jax.numpy as jnp
from jax import lax
from jax.experimental import pallas as pl
from jax.experimental.pallas import tpu as pltpu
from jax.experimental.pallas import tpu_sc as plsc

N = 10000
E = 160000
D = 256
NW = 32
NPT = 313
NPAD = NW * NPT
CAP = 16384
CHK = 8000
EB = 80

_mesh = plsc.VectorSubcoreMesh(core_axis_name="c", subcore_axis_name="s")
_sc_params = pltpu.CompilerParams(needs_layout_passes=False)


def _wid():
    return lax.axis_index("s") * 2 + lax.axis_index("c")


@functools.partial(
    pl.kernel,
    out_type=(
        jax.ShapeDtypeStruct((NW * CAP,), jnp.int32),
        jax.ShapeDtypeStruct((NW * CAP,), jnp.int32),
        jax.ShapeDtypeStruct((NW * CAP,), jnp.float32),
        jax.ShapeDtypeStruct((NW * 16,), jnp.int32),
    ),
    mesh=_mesh,
    scratch_types=[
        pltpu.VMEM((CHK,), jnp.int32),
        pltpu.VMEM((CHK,), jnp.int32),
        pltpu.VMEM((CHK,), jnp.float32),
        pltpu.VMEM((CAP + 16,), jnp.int32),
        pltpu.VMEM((CAP + 16,), jnp.int32),
        pltpu.VMEM((CAP + 16,), jnp.float32),
        pltpu.VMEM((CAP + 16,), jnp.int32),
        pltpu.VMEM((CAP + 16,), jnp.int32),
        pltpu.VMEM((CAP + 16,), jnp.float32),
        pltpu.VMEM((336,), jnp.int32),
        pltpu.VMEM((336,), jnp.int32),
        pltpu.VMEM((16,), jnp.int32),
    ],
    compiler_params=_sc_params,
)
def _route(src, dst, dist, src_o, off_o, dist_o, cnt_o,
           dst_c, src_c, dist_c, lsrc, loff, ldist, ssrc, soff, sdist,
           hist, start, cvec):
    wid = _wid()
    lo = wid * NPT
    iota = lax.iota(jnp.int32, 16)

    def chunk(c, ptr):
        base = c * CHK
        pltpu.sync_copy(dst.at[pl.ds(base, CHK)], dst_c)
        pltpu.sync_copy(src.at[pl.ds(base, CHK)], src_c)
        pltpu.sync_copy(dist.at[pl.ds(base, CHK)], dist_c)

        def group(j, p):
            d = dst_c[pl.ds(j * 16, 16)]
            m = (d >= lo) & (d < lo + NPT)
            key = jnp.where(m, iota, iota + 16)
            packed = src_c[pl.ds(j * 16, 16)] * 512 + (d - lo)
            _, po = plsc.sort_key_val(key, packed)
            _, di = plsc.sort_key_val(key, dist_c[pl.ds(j * 16, 16)])
            pos = jnp.minimum(p + iota, CAP + 15)
            plsc.store_scatter(lsrc, [pos], lax.shift_right_logical(po, 9))
            plsc.store_scatter(loff, [pos], po & 511)
            plsc.store_scatter(ldist, [pos], di)
            cnt = plsc.all_reduce_population_count(m)[0]
            return p + cnt

        return lax.fori_loop(0, CHK // 16, group, ptr, unroll=2)

    ptr = lax.fori_loop(0, E // CHK, chunk, jnp.int32(0))
    n = jnp.minimum(ptr, CAP - 64)

    zero16 = jnp.zeros((16,), jnp.int32)
    for g in range(336 // 16):
        hist[pl.ds(g * 16, 16)] = zero16
    onehot = jnp.where(iota == 0, 1, 0).astype(jnp.int32)

    def bhist(i, _):
        o = loff[pl.ds(i, 16)][0]
        plsc.addupdate(hist.at[pl.ds(o, 16)], onehot)
        return 0

    lax.fori_loop(0, n, bhist, 0)

    def bpre(g, carry):
        v = hist[pl.ds(g * 16, 16)]
        c = plsc.cumsum(v)
        start[pl.ds(g * 16, 16)] = c - v + carry
        return carry + c[15]

    lax.fori_loop(0, 336 // 16, bpre, jnp.int32(0))

    def bscat(i, _):
        o = loff[pl.ds(i, 16)][0]
        sv = start[pl.ds(o, 16)]
        p = sv[0]
        s = lsrc[pl.ds(i, 16)][0]
        dd = ldist[pl.ds(i, 16)][0]
        vs = ssrc[pl.ds(p, 16)]
        ssrc[pl.ds(p, 16)] = jnp.where(iota == 0, s, vs)
        vo = soff[pl.ds(p, 16)]
        soff[pl.ds(p, 16)] = jnp.where(iota == 0, o * D, vo)
        vd = sdist[pl.ds(p, 16)]
        sdist[pl.ds(p, 16)] = jnp.where(iota == 0, dd, vd)
        plsc.addupdate(start.at[pl.ds(o, 16)], onehot)
        return 0

    lax.fori_loop(0, n, bscat, 0)

    for g in range(128 // 16):
        pos = jnp.minimum(n + g * 16 + iota, CAP + 15)
        plsc.store_scatter(ssrc, [pos], jnp.zeros((16,), jnp.int32))
        plsc.store_scatter(soff, [pos], jnp.full((16,), NPT * D, jnp.int32))
        plsc.store_scatter(sdist, [pos], jnp.zeros((16,), jnp.float32))
    pltpu.sync_copy(ssrc.at[pl.ds(0, CAP)], src_o.at[pl.ds(wid * CAP, CAP)])
    pltpu.sync_copy(soff.at[pl.ds(0, CAP)], off_o.at[pl.ds(wid * CAP, CAP)])
    pltpu.sync_copy(sdist.at[pl.ds(0, CAP)], dist_o.at[pl.ds(wid * CAP, CAP)])
    cvec[...] = jnp.zeros((16,), jnp.int32) + n
    pltpu.sync_copy(cvec, cnt_o.at[pl.ds(wid * 16, 16)])


@functools.partial(
    pl.kernel,
    out_type=jax.ShapeDtypeStruct((NPAD * D,), jnp.float32),
    mesh=_mesh,
    scratch_types=[
        pltpu.VMEM(((NPT + 1) * D,), jnp.float32),
        pltpu.VMEM((D,), jnp.float32),
        pltpu.VMEM((2, EB), jnp.int32),
        pltpu.VMEM((2, EB + 16), jnp.int32),
        pltpu.VMEM((2, EB + 16), jnp.float32),
        pltpu.VMEM((2, EB, D), jnp.float32),
        pltpu.VMEM((16,), jnp.int32),
        pltpu.VMEM((D,), jnp.float32),
        pltpu.VMEM((16,), jnp.int32),
        pltpu.SemaphoreType.DMA,
        pltpu.SemaphoreType.DMA,
        pltpu.SemaphoreType.DMA,
        pltpu.SemaphoreType.DMA,
    ],
    compiler_params=_sc_params,
)
def _edge(a_hbm, wrow, src_l, off_l, dist_l, cnt_l, out,
          acc, wv, idx, offc, dsc, rows, cvec, rbuf, pvbuf,
          sl0, sl1, sg0, sg1):
    wid = _wid()
    seml = (sl0, sl1)
    semg = (sg0, sg1)

    def fill(i, _):
        acc[pl.ds(i * 16, 16)] = jnp.full((16,), jnp.inf, jnp.float32)
        return 0

    lax.fori_loop(0, (NPT + 1) * D // 16, fill, 0)
    pltpu.sync_copy(wrow, wv)
    pltpu.sync_copy(cnt_l.at[pl.ds(wid * 16, 16)], cvec)
    n = jnp.minimum(cvec[...][0], CAP - 64)
    nb = (n + (EB - 1)) // EB

    def issue_l(b, par):
        pltpu.async_copy(src_l.at[pl.ds(wid * CAP + b * EB, EB)], idx.at[par],
                         seml[par])
        pltpu.async_copy(off_l.at[pl.ds(wid * CAP + b * EB, EB)],
                         offc.at[par, pl.ds(0, EB)], seml[par])
        pltpu.async_copy(dist_l.at[pl.ds(wid * CAP + b * EB, EB)],
                         dsc.at[par, pl.ds(0, EB)], seml[par])

    def wait_l(b, par):
        pltpu.make_async_copy(src_l.at[pl.ds(wid * CAP + b * EB, EB)], idx.at[par],
                              seml[par]).wait()
        pltpu.make_async_copy(off_l.at[pl.ds(wid * CAP + b * EB, EB)],
                              offc.at[par, pl.ds(0, EB)], seml[par]).wait()
        pltpu.make_async_copy(dist_l.at[pl.ds(wid * CAP + b * EB, EB)],
                              dsc.at[par, pl.ds(0, EB)], seml[par]).wait()

    def issue_g(par):
        pltpu.async_copy(a_hbm.at[idx.at[par]], rows.at[par], semg[par])

    def wait_g(par):
        pltpu.make_async_copy(a_hbm.at[idx.at[par]], rows.at[par],
                              semg[par]).wait()

    @pl.when(nb > 0)
    def _():
        issue_l(0, 0)
        wait_l(0, 0)
        issue_g(0)

    @pl.when(nb > 1)
    def _():
        issue_l(1, 1)

    wvs = tuple(wv[pl.ds(g * 16, 16)] for g in range(D // 16))
    iota = lax.iota(jnp.int32, 16)
    inf16 = jnp.full((16,), jnp.inf, jnp.float32)
    pvbuf[...] = jnp.full((16,), -1, jnp.int32)
    zf = jnp.zeros((16,), jnp.float32)
    for g in range(D // 16):
        rbuf[pl.ds(g * 16, 16)] = zf

    def pair(p, carry):
        for q in range(2):
            par = q
            nxt = 1 - q
            bi = p * 2 + q

            @pl.when(bi < nb)
            def _():
                wait_g(par)

                @pl.when(bi + 1 < nb)
                def _():
                    wait_l(bi + 1, nxt)
                    issue_g(nxt)

                po0 = pvbuf[...]
                rm0 = tuple(rbuf[pl.ds(g * 16, 16)] for g in range(D // 16))

                def edge(ip, st):
                    po, rm = st
                    i0 = ip * 2
                    i1 = i0 + 1
                    gb = (i0 // 16) * 16
                    j0 = i0 - gb
                    offv = offc[par, pl.ds(gb, 16)]
                    distv = dsc[par, pl.ds(gb, 16)]
                    jv0 = jnp.zeros((16,), jnp.int32) + j0
                    jv1 = jv0 + 1
                    b0 = offv.at[jv0].get(mode="promise_in_bounds")
                    d0 = distv.at[jv0].get(mode="promise_in_bounds")
                    b1 = offv.at[jv1].get(mode="promise_in_bounds")
                    d1 = distv.at[jv1].get(mode="promise_in_bounds")
                    m0 = b0 == po
                    m1 = b1 == b0
                    a0 = b0 + iota
                    a1 = b1 + iota
                    nm = []
                    for g in range(D // 16):
                        r0 = rows[par, i0, pl.ds(g * 16, 16)]
                        r1 = rows[par, i1, pl.ds(g * 16, 16)]
                        e0 = r0 + d0 * carry[g]
                        e1 = r1 + d1 * carry[g]
                        v0 = jnp.minimum(e0, jnp.where(m0, rm[g], inf16))
                        v1 = jnp.minimum(e1, jnp.where(m1, v0, inf16))
                        plsc.store_scatter(acc, [a0 + (g * 16)], v0)
                        plsc.store_scatter(acc, [a1 + (g * 16)], v1)
                        nm.append(v1)
                    return (b1, tuple(nm))

                po1, rm1 = lax.fori_loop(0, EB // 2, edge, (po0, rm0))
                pvbuf[...] = po1
                for g in range(D // 16):
                    rbuf[pl.ds(g * 16, 16)] = rm1[g]

                @pl.when(bi + 2 < nb)
                def _():
                    issue_l(bi + 2, par)

        return carry

    lax.fori_loop(0, (nb + 1) // 2, pair, wvs)
    pltpu.sync_copy(acc.at[pl.ds(0, NPT * D)],
                    out.at[pl.ds(wid * NPT * D, NPT * D)])


_BLK = 2000


def _leaky(x):
    return jnp.where(x > 0, x, 0.01 * x)


def _project_body(h_ref, w_ref, o_ref):
    o_ref[...] = jnp.dot(h_ref[...], w_ref[...],
                         preferred_element_type=jnp.float32)


def _project(h, w):
    return pl.pallas_call(
        _project_body,
        grid=(N // _BLK,),
        in_specs=[
            pl.BlockSpec((_BLK, D), lambda i: (i, 0)),
            pl.BlockSpec((D, D), lambda i: (0, 0)),
        ],
        out_specs=pl.BlockSpec((_BLK, D), lambda i: (i, 0)),
        out_shape=jax.ShapeDtypeStruct((N, D), jnp.float32),
    )(h, w)


def _node_body(h_ref, s_ref, wd_ref, wn_ref, o_ref):
    h = h_ref[...]
    t = s_ref[...] + jnp.dot(h, wd_ref[...], preferred_element_type=jnp.float32)
    red = jnp.where(jnp.isfinite(t), _leaky(t), 0.0)
    wn = wn_ref[...]
    o = (jnp.dot(h, wn[:D], preferred_element_type=jnp.float32)
         + jnp.dot(red, wn[D:], preferred_element_type=jnp.float32))
    o_ref[...] = _leaky(o)


def _node(h, s, wd, wn):
    return pl.pallas_call(
        _node_body,
        grid=(N // _BLK,),
        in_specs=[
            pl.BlockSpec((_BLK, D), lambda i: (i, 0)),
            pl.BlockSpec((_BLK, D), lambda i: (i, 0)),
            pl.BlockSpec((D, D), lambda i: (0, 0)),
            pl.BlockSpec((2 * D, D), lambda i: (0, 0)),
        ],
        out_specs=pl.BlockSpec((_BLK, D), lambda i: (i, 0)),
        out_shape=jax.ShapeDtypeStruct((N, D), jnp.float32),
    )(h, s, wd, wn)


def kernel(nf, edge_index, dist, We0, Wn0, We1, Wn1, We2, Wn2):
    src_l, off_l, dist_l, cnt_l = _route(edge_index[0], edge_index[1], dist)
    h = nf
    for We, Wn in ((We0, Wn0), (We1, Wn1), (We2, Wn2)):
        a = _project(h, We[1:D + 1])
        s_flat = _edge(a, We[0], src_l, off_l, dist_l, cnt_l)
        s = s_flat.reshape(NPAD, D)[:N]
        h = _node(h, s, We[D + 1:], Wn)
    return h


# speedup vs baseline: 1.5141x; 1.1566x over previous
import functools

import jax
import jax.numpy as jnp
from jax import lax
from jax.experimental import pallas as pl
from jax.experimental.pallas import tpu as pltpu
from jax.experimental.pallas import tpu_sc as plsc

N = 10000
E = 160000
D = 256
NW = 32
NPT = 313
NPAD = NW * NPT
CAP = 16384
CHK = 8000
EB = 80

_mesh = plsc.VectorSubcoreMesh(core_axis_name="c", subcore_axis_name="s")
_sc_params = pltpu.CompilerParams(needs_layout_passes=False)


def _wid():
    return lax.axis_index("s") * 2 + lax.axis_index("c")


@functools.partial(
    pl.kernel,
    out_type=(
        jax.ShapeDtypeStruct((NW * CAP,), jnp.int32),
        jax.ShapeDtypeStruct((NW * CAP,), jnp.int32),
        jax.ShapeDtypeStruct((NW * CAP,), jnp.float32),
        jax.ShapeDtypeStruct((NW * 16,), jnp.int32),
    ),
    mesh=_mesh,
    scratch_types=[
        pltpu.VMEM((CHK,), jnp.int32),
        pltpu.VMEM((CHK,), jnp.int32),
        pltpu.VMEM((CHK,), jnp.float32),
        pltpu.VMEM((CAP + 16,), jnp.int32),
        pltpu.VMEM((CAP + 16,), jnp.int32),
        pltpu.VMEM((CAP + 16,), jnp.float32),
        pltpu.VMEM((CAP + 16,), jnp.int32),
        pltpu.VMEM((CAP + 16,), jnp.int32),
        pltpu.VMEM((CAP + 16,), jnp.float32),
        pltpu.VMEM((336,), jnp.int32),
        pltpu.VMEM((336,), jnp.int32),
        pltpu.VMEM((16,), jnp.int32),
    ],
    compiler_params=_sc_params,
)
def _route(src, dst, dist, src_o, off_o, dist_o, cnt_o,
           dst_c, src_c, dist_c, lsrc, loff, ldist, ssrc, soff, sdist,
           hist, start, cvec):
    wid = _wid()
    lo = wid * NPT
    iota = lax.iota(jnp.int32, 16)

    def chunk(c, ptr):
        base = c * CHK
        pltpu.sync_copy(dst.at[pl.ds(base, CHK)], dst_c)
        pltpu.sync_copy(src.at[pl.ds(base, CHK)], src_c)
        pltpu.sync_copy(dist.at[pl.ds(base, CHK)], dist_c)

        def group(j, p):
            d = dst_c[pl.ds(j * 16, 16)]
            m = (d >= lo) & (d < lo + NPT)
            key = jnp.where(m, iota, iota + 16)
            packed = src_c[pl.ds(j * 16, 16)] * 512 + (d - lo)
            _, po = plsc.sort_key_val(key, packed)
            _, di = plsc.sort_key_val(key, dist_c[pl.ds(j * 16, 16)])
            pos = jnp.minimum(p + iota, CAP + 15)
            plsc.store_scatter(lsrc, [pos], lax.shift_right_logical(po, 9))
            plsc.store_scatter(loff, [pos], po & 511)
            plsc.store_scatter(ldist, [pos], di)
            cnt = plsc.all_reduce_population_count(m)[0]
            return p + cnt

        return lax.fori_loop(0, CHK // 16, group, ptr, unroll=2)

    ptr = lax.fori_loop(0, E // CHK, chunk, jnp.int32(0))
    n = jnp.minimum(ptr, CAP - 64)

    zero16 = jnp.zeros((16,), jnp.int32)
    for g in range(336 // 16):
        hist[pl.ds(g * 16, 16)] = zero16
    onehot = jnp.where(iota == 0, 1, 0).astype(jnp.int32)

    def bhist(i, _):
        o = loff[pl.ds(i, 16)][0]
        plsc.addupdate(hist.at[pl.ds(o, 16)], onehot)
        return 0

    lax.fori_loop(0, n, bhist, 0)

    def bpre(g, carry):
        v = hist[pl.ds(g * 16, 16)]
        c = plsc.cumsum(v)
        start[pl.ds(g * 16, 16)] = c - v + carry
        return carry + c[15]

    lax.fori_loop(0, 336 // 16, bpre, jnp.int32(0))

    def bscat(i, _):
        o = loff[pl.ds(i, 16)][0]
        sv = start[pl.ds(o, 16)]
        p = sv[0]
        s = lsrc[pl.ds(i, 16)][0]
        dd = ldist[pl.ds(i, 16)][0]
        vs = ssrc[pl.ds(p, 16)]
        ssrc[pl.ds(p, 16)] = jnp.where(iota == 0, s, vs)
        vo = soff[pl.ds(p, 16)]
        soff[pl.ds(p, 16)] = jnp.where(iota == 0, o * D, vo)
        vd = sdist[pl.ds(p, 16)]
        sdist[pl.ds(p, 16)] = jnp.where(iota == 0, dd, vd)
        plsc.addupdate(start.at[pl.ds(o, 16)], onehot)
        return 0

    lax.fori_loop(0, n, bscat, 0)

    for g in range(128 // 16):
        pos = jnp.minimum(n + g * 16 + iota, CAP + 15)
        plsc.store_scatter(ssrc, [pos], jnp.zeros((16,), jnp.int32))
        plsc.store_scatter(soff, [pos], jnp.full((16,), NPT * D, jnp.int32))
        plsc.store_scatter(sdist, [pos], jnp.zeros((16,), jnp.float32))
    pltpu.sync_copy(ssrc.at[pl.ds(0, CAP)], src_o.at[pl.ds(wid * CAP, CAP)])
    pltpu.sync_copy(soff.at[pl.ds(0, CAP)], off_o.at[pl.ds(wid * CAP, CAP)])
    pltpu.sync_copy(sdist.at[pl.ds(0, CAP)], dist_o.at[pl.ds(wid * CAP, CAP)])
    cvec[...] = jnp.zeros((16,), jnp.int32) + n
    pltpu.sync_copy(cvec, cnt_o.at[pl.ds(wid * 16, 16)])


@functools.partial(
    pl.kernel,
    out_type=jax.ShapeDtypeStruct((NPAD * D,), jnp.float32),
    mesh=_mesh,
    scratch_types=[
        pltpu.VMEM(((NPT + 1) * D,), jnp.float32),
        pltpu.VMEM((D,), jnp.float32),
        pltpu.VMEM((2, EB), jnp.int32),
        pltpu.VMEM((2, EB + 16), jnp.int32),
        pltpu.VMEM((2, EB + 16), jnp.float32),
        pltpu.VMEM((2, EB, D), jnp.float32),
        pltpu.VMEM((16,), jnp.int32),
        pltpu.VMEM((D,), jnp.float32),
        pltpu.VMEM((16,), jnp.int32),
        pltpu.SemaphoreType.DMA,
        pltpu.SemaphoreType.DMA,
        pltpu.SemaphoreType.DMA,
        pltpu.SemaphoreType.DMA,
    ],
    compiler_params=_sc_params,
)
def _edge(a_hbm, wrow, src_l, off_l, dist_l, cnt_l, out,
          acc, wv, idx, offc, dsc, rows, cvec, rbuf, pvbuf,
          sl0, sl1, sg0, sg1):
    wid = _wid()
    seml = (sl0, sl1)
    semg = (sg0, sg1)

    def fill(i, _):
        acc[pl.ds(i * 16, 16)] = jnp.full((16,), jnp.inf, jnp.float32)
        return 0

    lax.fori_loop(0, (NPT + 1) * D // 16, fill, 0)
    pltpu.sync_copy(wrow, wv)
    pltpu.sync_copy(cnt_l.at[pl.ds(wid * 16, 16)], cvec)
    n = jnp.minimum(cvec[...][0], CAP - 64)
    nb = (n + (EB - 1)) // EB

    def issue_l(b, par):
        pltpu.async_copy(src_l.at[pl.ds(wid * CAP + b * EB, EB)], idx.at[par],
                         seml[par])
        pltpu.async_copy(off_l.at[pl.ds(wid * CAP + b * EB, EB)],
                         offc.at[par, pl.ds(0, EB)], seml[par])
        pltpu.async_copy(dist_l.at[pl.ds(wid * CAP + b * EB, EB)],
                         dsc.at[par, pl.ds(0, EB)], seml[par])

    def wait_l(b, par):
        pltpu.make_async_copy(src_l.at[pl.ds(wid * CAP + b * EB, EB)], idx.at[par],
                              seml[par]).wait()
        pltpu.make_async_copy(off_l.at[pl.ds(wid * CAP + b * EB, EB)],
                              offc.at[par, pl.ds(0, EB)], seml[par]).wait()
        pltpu.make_async_copy(dist_l.at[pl.ds(wid * CAP + b * EB, EB)],
                              dsc.at[par, pl.ds(0, EB)], seml[par]).wait()

    def issue_g(par):
        pltpu.async_copy(a_hbm.at[idx.at[par]], rows.at[par], semg[par])

    def wait_g(par):
        pltpu.make_async_copy(a_hbm.at[idx.at[par]], rows.at[par],
                              semg[par]).wait()

    @pl.when(nb > 0)
    def _():
        issue_l(0, 0)
        wait_l(0, 0)
        issue_g(0)

    @pl.when(nb > 1)
    def _():
        issue_l(1, 1)

    wvs = tuple(wv[pl.ds(g * 16, 16)] for g in range(D // 16))
    iota = lax.iota(jnp.int32, 16)
    inf16 = jnp.full((16,), jnp.inf, jnp.float32)
    pvbuf[...] = jnp.full((16,), -1, jnp.int32)
    zf = jnp.zeros((16,), jnp.float32)
    for g in range(D // 16):
        rbuf[pl.ds(g * 16, 16)] = zf

    def pair(p, carry):
        for q in range(2):
            par = q
            nxt = 1 - q
            bi = p * 2 + q

            @pl.when(bi < nb)
            def _():
                wait_g(par)

                @pl.when(bi + 1 < nb)
                def _():
                    wait_l(bi + 1, nxt)
                    issue_g(nxt)

                po0 = pvbuf[...]
                rm0 = tuple(rbuf[pl.ds(g * 16, 16)] for g in range(D // 16))

                U = 4

                def edge(ip, st):
                    po, rm = st
                    i0 = ip * U
                    gb = (i0 // 16) * 16
                    j0 = i0 - gb
                    offv = offc[par, pl.ds(gb, 16)]
                    distv = dsc[par, pl.ds(gb, 16)]
                    jv = [jnp.zeros((16,), jnp.int32) + (j0 + u)
                          for u in range(U)]
                    bs = [offv.at[j].get(mode="promise_in_bounds") for j in jv]
                    dv = [distv.at[j].get(mode="promise_in_bounds") for j in jv]
                    ms = [bs[0] == po] + [bs[u] == bs[u - 1]
                                          for u in range(1, U)]
                    av = [b + iota for b in bs]
                    nm = []
                    for g in range(D // 16):
                        rs = [rows[par, i0 + u, pl.ds(g * 16, 16)]
                              for u in range(U)]
                        es = [rs[u] + dv[u] * carry[g] for u in range(U)]
                        vp = rm[g]
                        for u in range(U):
                            vp = jnp.minimum(es[u],
                                             jnp.where(ms[u], vp, inf16))
                            plsc.store_scatter(acc, [av[u] + (g * 16)], vp)
                        nm.append(vp)
                    return (bs[U - 1], tuple(nm))

                po1, rm1 = lax.fori_loop(0, EB // U, edge, (po0, rm0))
                pvbuf[...] = po1
                for g in range(D // 16):
                    rbuf[pl.ds(g * 16, 16)] = rm1[g]

                @pl.when(bi + 2 < nb)
                def _():
                    issue_l(bi + 2, par)

        return carry

    lax.fori_loop(0, (nb + 1) // 2, pair, wvs)
    pltpu.sync_copy(acc.at[pl.ds(0, NPT * D)],
                    out.at[pl.ds(wid * NPT * D, NPT * D)])


_BLK = 2000


def _leaky(x):
    return jnp.where(x > 0, x, 0.01 * x)


def _project_body(h_ref, w_ref, o_ref):
    o_ref[...] = jnp.dot(h_ref[...], w_ref[...],
                         preferred_element_type=jnp.float32)


def _project(h, w):
    return pl.pallas_call(
        _project_body,
        grid=(N // _BLK,),
        in_specs=[
            pl.BlockSpec((_BLK, D), lambda i: (i, 0)),
            pl.BlockSpec((D, D), lambda i: (0, 0)),
        ],
        out_specs=pl.BlockSpec((_BLK, D), lambda i: (i, 0)),
        out_shape=jax.ShapeDtypeStruct((N, D), jnp.float32),
    )(h, w)


def _node_body(h_ref, s_ref, wd_ref, wn_ref, o_ref):
    h = h_ref[...]
    t = s_ref[...] + jnp.dot(h, wd_ref[...], preferred_element_type=jnp.float32)
    red = jnp.where(jnp.isfinite(t), _leaky(t), 0.0)
    wn = wn_ref[...]
    o = (jnp.dot(h, wn[:D], preferred_element_type=jnp.float32)
         + jnp.dot(red, wn[D:], preferred_element_type=jnp.float32))
    o_ref[...] = _leaky(o)


def _node(h, s, wd, wn):
    return pl.pallas_call(
        _node_body,
        grid=(N // _BLK,),
        in_specs=[
            pl.BlockSpec((_BLK, D), lambda i: (i, 0)),
            pl.BlockSpec((_BLK, D), lambda i: (i, 0)),
            pl.BlockSpec((D, D), lambda i: (0, 0)),
            pl.BlockSpec((2 * D, D), lambda i: (0, 0)),
        ],
        out_specs=pl.BlockSpec((_BLK, D), lambda i: (i, 0)),
        out_shape=jax.ShapeDtypeStruct((N, D), jnp.float32),
    )(h, s, wd, wn)


def kernel(nf, edge_index, dist, We0, Wn0, We1, Wn1, We2, Wn2):
    src_l, off_l, dist_l, cnt_l = _route(edge_index[0], edge_index[1], dist)
    h = nf
    for We, Wn in ((We0, Wn0), (We1, Wn1), (We2, Wn2)):
        a = _project(h, We[1:D + 1])
        s_flat = _edge(a, We[0], src_l, off_l, dist_l, cnt_l)
        s = s_flat.reshape(NPAD, D)[:N]
        h = _node(h, s, We[D + 1:], Wn)
    return h


# speedup vs baseline: 1.6712x; 1.1037x over previous
import functools

import jax
import jax.numpy as jnp
from jax import lax
from jax.experimental import pallas as pl
from jax.experimental.pallas import tpu as pltpu
from jax.experimental.pallas import tpu_sc as plsc

N = 10000
E = 160000
D = 256
NW = 32
NPT = 313
NPAD = NW * NPT
CAP = 16384
CHK = 8000
EB = 80

_mesh = plsc.VectorSubcoreMesh(core_axis_name="c", subcore_axis_name="s")
_sc_params = pltpu.CompilerParams(needs_layout_passes=False)


def _wid():
    return lax.axis_index("s") * 2 + lax.axis_index("c")


@functools.partial(
    pl.kernel,
    out_type=(
        jax.ShapeDtypeStruct((NW * CAP,), jnp.int32),
        jax.ShapeDtypeStruct((NW * CAP,), jnp.int32),
        jax.ShapeDtypeStruct((NW * CAP,), jnp.float32),
        jax.ShapeDtypeStruct((NW * 16,), jnp.int32),
    ),
    mesh=_mesh,
    scratch_types=[
        pltpu.VMEM((CHK,), jnp.int32),
        pltpu.VMEM((CHK,), jnp.int32),
        pltpu.VMEM((CHK,), jnp.float32),
        pltpu.VMEM((CAP + 16,), jnp.int32),
        pltpu.VMEM((CAP + 16,), jnp.int32),
        pltpu.VMEM((CAP + 16,), jnp.float32),
        pltpu.VMEM((CAP + 16,), jnp.int32),
        pltpu.VMEM((CAP + 16,), jnp.int32),
        pltpu.VMEM((CAP + 16,), jnp.float32),
        pltpu.VMEM((336,), jnp.int32),
        pltpu.VMEM((336,), jnp.int32),
        pltpu.VMEM((16,), jnp.int32),
    ],
    compiler_params=_sc_params,
)
def _route(src, dst, dist, src_o, off_o, dist_o, cnt_o,
           dst_c, src_c, dist_c, lsrc, loff, ldist, ssrc, soff, sdist,
           hist, start, cvec):
    wid = _wid()
    lo = wid * NPT
    iota = lax.iota(jnp.int32, 16)

    def chunk(c, ptr):
        base = c * CHK
        pltpu.sync_copy(dst.at[pl.ds(base, CHK)], dst_c)
        pltpu.sync_copy(src.at[pl.ds(base, CHK)], src_c)
        pltpu.sync_copy(dist.at[pl.ds(base, CHK)], dist_c)

        def group(j, p):
            d = dst_c[pl.ds(j * 16, 16)]
            m = (d >= lo) & (d < lo + NPT)
            key = jnp.where(m, iota, iota + 16)
            packed = src_c[pl.ds(j * 16, 16)] * 512 + (d - lo)
            _, po = plsc.sort_key_val(key, packed)
            _, di = plsc.sort_key_val(key, dist_c[pl.ds(j * 16, 16)])
            pos = jnp.minimum(p + iota, CAP + 15)
            plsc.store_scatter(lsrc, [pos], lax.shift_right_logical(po, 9))
            plsc.store_scatter(loff, [pos], po & 511)
            plsc.store_scatter(ldist, [pos], di)
            cnt = plsc.all_reduce_population_count(m)[0]
            return p + cnt

        return lax.fori_loop(0, CHK // 16, group, ptr, unroll=2)

    ptr = lax.fori_loop(0, E // CHK, chunk, jnp.int32(0))
    n = jnp.minimum(ptr, CAP - 64)

    zero16 = jnp.zeros((16,), jnp.int32)
    for g in range(336 // 16):
        hist[pl.ds(g * 16, 16)] = zero16
    onehot = jnp.where(iota == 0, 1, 0).astype(jnp.int32)

    def bhist(i, _):
        o = loff[pl.ds(i, 16)][0]
        plsc.addupdate(hist.at[pl.ds(o, 16)], onehot)
        return 0

    lax.fori_loop(0, n, bhist, 0)

    def bpre(g, carry):
        v = hist[pl.ds(g * 16, 16)]
        c = plsc.cumsum(v)
        start[pl.ds(g * 16, 16)] = c - v + carry
        return carry + c[15]

    lax.fori_loop(0, 336 // 16, bpre, jnp.int32(0))

    def bscat(i, _):
        o = loff[pl.ds(i, 16)][0]
        sv = start[pl.ds(o, 16)]
        p = sv[0]
        s = lsrc[pl.ds(i, 16)][0]
        dd = ldist[pl.ds(i, 16)][0]
        vs = ssrc[pl.ds(p, 16)]
        ssrc[pl.ds(p, 16)] = jnp.where(iota == 0, s, vs)
        vo = soff[pl.ds(p, 16)]
        soff[pl.ds(p, 16)] = jnp.where(iota == 0, o * D, vo)
        vd = sdist[pl.ds(p, 16)]
        sdist[pl.ds(p, 16)] = jnp.where(iota == 0, dd, vd)
        plsc.addupdate(start.at[pl.ds(o, 16)], onehot)
        return 0

    lax.fori_loop(0, n, bscat, 0)

    for g in range(128 // 16):
        pos = jnp.minimum(n + g * 16 + iota, CAP + 15)
        plsc.store_scatter(ssrc, [pos], jnp.zeros((16,), jnp.int32))
        plsc.store_scatter(soff, [pos], jnp.full((16,), NPT * D, jnp.int32))
        plsc.store_scatter(sdist, [pos], jnp.zeros((16,), jnp.float32))
    pltpu.sync_copy(ssrc.at[pl.ds(0, CAP)], src_o.at[pl.ds(wid * CAP, CAP)])
    pltpu.sync_copy(soff.at[pl.ds(0, CAP)], off_o.at[pl.ds(wid * CAP, CAP)])
    pltpu.sync_copy(sdist.at[pl.ds(0, CAP)], dist_o.at[pl.ds(wid * CAP, CAP)])
    cvec[...] = jnp.zeros((16,), jnp.int32) + n
    pltpu.sync_copy(cvec, cnt_o.at[pl.ds(wid * 16, 16)])


@functools.partial(
    pl.kernel,
    out_type=jax.ShapeDtypeStruct((NPAD * D,), jnp.float32),
    mesh=_mesh,
    scratch_types=[
        pltpu.VMEM(((NPT + 1) * D,), jnp.float32),
        pltpu.VMEM((D,), jnp.float32),
        pltpu.VMEM((2, EB), jnp.int32),
        pltpu.VMEM((2, EB + 16), jnp.int32),
        pltpu.VMEM((2, EB + 16), jnp.float32),
        pltpu.VMEM((2, EB, D), jnp.float32),
        pltpu.VMEM((16,), jnp.int32),
        pltpu.VMEM((D,), jnp.float32),
        pltpu.VMEM((16,), jnp.int32),
        pltpu.SemaphoreType.DMA,
        pltpu.SemaphoreType.DMA,
        pltpu.SemaphoreType.DMA,
        pltpu.SemaphoreType.DMA,
    ],
    compiler_params=_sc_params,
)
def _edge(a_hbm, wrow, src_l, off_l, dist_l, cnt_l, out,
          acc, wv, idx, offc, dsc, rows, cvec, rbuf, pvbuf,
          sl0, sl1, sg0, sg1):
    wid = _wid()
    seml = (sl0, sl1)
    semg = (sg0, sg1)

    def fill(i, _):
        acc[pl.ds(i * 16, 16)] = jnp.full((16,), jnp.inf, jnp.float32)
        return 0

    lax.fori_loop(0, (NPT + 1) * D // 16, fill, 0)
    pltpu.sync_copy(wrow, wv)
    pltpu.sync_copy(cnt_l.at[pl.ds(wid * 16, 16)], cvec)
    n = jnp.minimum(cvec[...][0], CAP - 64)
    nb = (n + (EB - 1)) // EB

    def issue_l(b, par):
        pltpu.async_copy(src_l.at[pl.ds(wid * CAP + b * EB, EB)], idx.at[par],
                         seml[par])
        pltpu.async_copy(off_l.at[pl.ds(wid * CAP + b * EB, EB)],
                         offc.at[par, pl.ds(0, EB)], seml[par])
        pltpu.async_copy(dist_l.at[pl.ds(wid * CAP + b * EB, EB)],
                         dsc.at[par, pl.ds(0, EB)], seml[par])

    def wait_l(b, par):
        pltpu.make_async_copy(src_l.at[pl.ds(wid * CAP + b * EB, EB)], idx.at[par],
                              seml[par]).wait()
        pltpu.make_async_copy(off_l.at[pl.ds(wid * CAP + b * EB, EB)],
                              offc.at[par, pl.ds(0, EB)], seml[par]).wait()
        pltpu.make_async_copy(dist_l.at[pl.ds(wid * CAP + b * EB, EB)],
                              dsc.at[par, pl.ds(0, EB)], seml[par]).wait()

    def issue_g(par):
        pltpu.async_copy(a_hbm.at[idx.at[par]], rows.at[par], semg[par])

    def wait_g(par):
        pltpu.make_async_copy(a_hbm.at[idx.at[par]], rows.at[par],
                              semg[par]).wait()

    @pl.when(nb > 0)
    def _():
        issue_l(0, 0)
        wait_l(0, 0)
        issue_g(0)

    @pl.when(nb > 1)
    def _():
        issue_l(1, 1)

    wvs = tuple(wv[pl.ds(g * 16, 16)] for g in range(D // 16))
    iota = lax.iota(jnp.int32, 16)
    inf16 = jnp.full((16,), jnp.inf, jnp.float32)
    pvbuf[...] = jnp.full((16,), -1, jnp.int32)
    zf = jnp.zeros((16,), jnp.float32)
    for g in range(D // 16):
        rbuf[pl.ds(g * 16, 16)] = zf

    def pair(p, carry):
        for q in range(2):
            par = q
            nxt = 1 - q
            bi = p * 2 + q

            @pl.when(bi < nb)
            def _():
                wait_g(par)

                @pl.when(bi + 1 < nb)
                def _():
                    wait_l(bi + 1, nxt)
                    issue_g(nxt)

                po0 = pvbuf[...]
                rm0 = tuple(rbuf[pl.ds(g * 16, 16)] for g in range(D // 16))

                U = 8

                def edge(ip, st):
                    po, rm = st
                    i0 = ip * U
                    gb = (i0 // 16) * 16
                    j0 = i0 - gb
                    offv = offc[par, pl.ds(gb, 16)]
                    distv = dsc[par, pl.ds(gb, 16)]
                    jv = [jnp.zeros((16,), jnp.int32) + (j0 + u)
                          for u in range(U)]
                    bs = [offv.at[j].get(mode="promise_in_bounds") for j in jv]
                    dv = [distv.at[j].get(mode="promise_in_bounds") for j in jv]
                    ms = [bs[0] == po] + [bs[u] == bs[u - 1]
                                          for u in range(1, U)]
                    av = [b + iota for b in bs]
                    nm = []
                    for g in range(D // 16):
                        rs = [rows[par, i0 + u, pl.ds(g * 16, 16)]
                              for u in range(U)]
                        es = [rs[u] + dv[u] * carry[g] for u in range(U)]
                        vp = rm[g]
                        for u in range(U):
                            vp = jnp.minimum(es[u],
                                             jnp.where(ms[u], vp, inf16))
                            plsc.store_scatter(acc, [av[u] + (g * 16)], vp)
                        nm.append(vp)
                    return (bs[U - 1], tuple(nm))

                po1, rm1 = lax.fori_loop(0, EB // U, edge, (po0, rm0))
                pvbuf[...] = po1
                for g in range(D // 16):
                    rbuf[pl.ds(g * 16, 16)] = rm1[g]

                @pl.when(bi + 2 < nb)
                def _():
                    issue_l(bi + 2, par)

        return carry

    lax.fori_loop(0, (nb + 1) // 2, pair, wvs)
    pltpu.sync_copy(acc.at[pl.ds(0, NPT * D)],
                    out.at[pl.ds(wid * NPT * D, NPT * D)])


_BLK = 2000


def _leaky(x):
    return jnp.where(x > 0, x, 0.01 * x)


def _project_body(h_ref, w_ref, o_ref):
    o_ref[...] = jnp.dot(h_ref[...], w_ref[...],
                         preferred_element_type=jnp.float32)


def _project(h, w):
    return pl.pallas_call(
        _project_body,
        grid=(N // _BLK,),
        in_specs=[
            pl.BlockSpec((_BLK, D), lambda i: (i, 0)),
            pl.BlockSpec((D, D), lambda i: (0, 0)),
        ],
        out_specs=pl.BlockSpec((_BLK, D), lambda i: (i, 0)),
        out_shape=jax.ShapeDtypeStruct((N, D), jnp.float32),
    )(h, w)


def _node_body(h_ref, s_ref, wd_ref, wn_ref, o_ref):
    h = h_ref[...]
    t = s_ref[...] + jnp.dot(h, wd_ref[...], preferred_element_type=jnp.float32)
    red = jnp.where(jnp.isfinite(t), _leaky(t), 0.0)
    wn = wn_ref[...]
    o = (jnp.dot(h, wn[:D], preferred_element_type=jnp.float32)
         + jnp.dot(red, wn[D:], preferred_element_type=jnp.float32))
    o_ref[...] = _leaky(o)


def _node(h, s, wd, wn):
    return pl.pallas_call(
        _node_body,
        grid=(N // _BLK,),
        in_specs=[
            pl.BlockSpec((_BLK, D), lambda i: (i, 0)),
            pl.BlockSpec((_BLK, D), lambda i: (i, 0)),
            pl.BlockSpec((D, D), lambda i: (0, 0)),
            pl.BlockSpec((2 * D, D), lambda i: (0, 0)),
        ],
        out_specs=pl.BlockSpec((_BLK, D), lambda i: (i, 0)),
        out_shape=jax.ShapeDtypeStruct((N, D), jnp.float32),
    )(h, s, wd, wn)


def kernel(nf, edge_index, dist, We0, Wn0, We1, Wn1, We2, Wn2):
    src_l, off_l, dist_l, cnt_l = _route(edge_index[0], edge_index[1], dist)
    h = nf
    for We, Wn in ((We0, Wn0), (We1, Wn1), (We2, Wn2)):
        a = _project(h, We[1:D + 1])
        s_flat = _edge(a, We[0], src_l, off_l, dist_l, cnt_l)
        s = s_flat.reshape(NPAD, D)[:N]
        h = _node(h, s, We[D + 1:], Wn)
    return h
